# baseline (speedup 1.0000x reference)
import functools

import jax
import jax.numpy as jnp
from jax import lax
from jax.experimental import pallas as pl
from jax.experimental.pallas import tpu as pltpu
from jax.experimental.pallas import tpu_sc as plsc

N = 10000
E = 320000
D = 128
H = 128

NC = 2
NS = 16
NW = NC * NS
LANES = 16

BLK = 128
BPT = 80
E_PAD = NW * BPT * BLK
NBLK = E_PAD // BLK

CH = 16
NBLK_ALLOC = NBLK + CH

BLK2 = 128
QD = 2
CH2 = 16
NBLK2 = E_PAD // BLK2
BPT2_0 = 160
BPT2_1 = 0

NP = 10240
SL1 = NP // NS
ROWS_PT = NP // NS

_mesh = plsc.VectorSubcoreMesh(core_axis_name="c", subcore_axis_name="s")


def _zero_vec(ref, n):
    def body(i, _):
        ref[pl.ds(i * LANES, LANES)] = jnp.zeros((LANES,), jnp.float32)
        return 0
    lax.fori_loop(0, n // LANES, body, 0, unroll=4)


def _deg_body(dst_hbm, ew_hbm, out_hbm, dstb, ewb, zb, acc):
    cid = lax.axis_index("c")
    sid = lax.axis_index("s")
    wid = sid * NC + cid

    _zero_vec(zb, SL1)
    pltpu.sync_copy(zb, acc.at[pl.ds(sid * SL1, SL1)])
    plsc.subcore_barrier()

    pltpu.sync_copy(dst_hbm.at[pl.ds(wid * BPT, BPT)], dstb)
    pltpu.sync_copy(ew_hbm.at[pl.ds(wid * BPT, BPT)], ewb)

    def body(b, _):
        pltpu.sync_copy(ewb.at[b], acc.at[dstb.at[b]], add=True)
        return 0
    lax.fori_loop(0, BPT, body, 0)

    plsc.subcore_barrier()
    pltpu.sync_copy(acc.at[pl.ds(sid * SL1, SL1)],
                    out_hbm.at[pl.ds(cid * NP + sid * SL1, SL1)])


_deg_kernel = functools.partial(
    pl.kernel,
    out_type=jax.ShapeDtypeStruct((NC * NP,), jnp.float32),
    mesh=_mesh,
    compiler_params=pltpu.CompilerParams(needs_layout_passes=False),
    scratch_types=[
        pltpu.VMEM((BPT, BLK), jnp.int32),
        pltpu.VMEM((BPT, BLK), jnp.float32),
        pltpu.VMEM((SL1,), jnp.float32),
        pltpu.VMEM_SHARED((NP,), jnp.float32),
    ],
)(_deg_body)


def _agg_body(g1a_hbm, src_hbm, dst_hbm, ew_hbm,
              outa, outb, srcb, dstb, ewb, rows, acc, gsem, ssem):
    cid = lax.axis_index("c")
    sid = lax.axis_index("s")
    nblocks = lax.select(cid == 0, BPT2_0, BPT2_1)
    blk_base = lax.select(cid == 0, sid * BPT2_0, NS * BPT2_0 + sid * BPT2_1)

    def zrow(r, _):
        for c in range(8):
            rows[r, pl.ds(c * LANES, LANES)] = jnp.zeros((LANES,), jnp.float32)
        return 0
    lax.fori_loop(0, 128, zrow, 0)
    base = sid * ROWS_PT
    for off in (0, 128, 256, 384, 512):
        pltpu.sync_copy(rows.at[pl.ds(0, 128)], acc.at[pl.ds(base + off, 128)])
    plsc.subcore_barrier()

    def start_gather(bi, q):
        pltpu.async_copy(g1a_hbm.at[srcb.at[pl.ds(bi * BLK2, BLK2)]],
                         rows.at[pl.ds(q * BLK2, BLK2)], gsem)

    def wait_gather():
        pltpu.make_async_copy(g1a_hbm.at[srcb.at[pl.ds(0, BLK2)]],
                              rows.at[pl.ds(0, BLK2)], gsem).wait()

    def start_scatter(bi, q):
        pltpu.async_copy(rows.at[pl.ds(q * BLK2, BLK2)],
                         acc.at[dstb.at[bi]], ssem, add=True)

    def wait_scatter():
        pltpu.make_async_copy(rows.at[pl.ds(0, BLK2)],
                              acc.at[dstb.at[0]], ssem).wait()

    def refill(ci):
        off = blk_base + ci * CH2
        pltpu.sync_copy(src_hbm.at[pl.ds(off * BLK2, CH2 * BLK2)], srcb)
        pltpu.sync_copy(dst_hbm.at[pl.ds(off, CH2)], dstb)
        pltpu.sync_copy(ew_hbm.at[pl.ds(off * BLK2, CH2 * BLK2)], ewb)

    def body(b, _):
        bi = lax.rem(b, CH2)
        q = lax.rem(b, QD)

        @pl.when(bi == 0)
        def _():
            @pl.when(b > 0)
            def _():
                wait_scatter()
            refill(lax.div(b, CH2))
            for p in range(QD - 1):
                @pl.when(b + p < nblocks)
                def _():
                    start_gather(p, lax.rem(b + p, QD))

        wait_gather()

        @pl.when((bi != 0) & (b > 0))
        def _():
            wait_scatter()

        @pl.when((bi + QD - 1 < CH2) & (b + QD - 1 < nblocks))
        def _():
            start_gather(bi + QD - 1, lax.rem(b + QD - 1, QD))

        def scale(rr, _):
            bc = plsc.load_gather(
                ewb, [jnp.full((LANES,), bi * BLK2 + rr, jnp.int32)])
            row = q * BLK2 + rr
            for c in range(8):
                sl = pl.ds(c * LANES, LANES)
                rows[row, sl] = rows[row, sl] * bc
            return 0
        lax.fori_loop(0, BLK2, scale, 0, unroll=4)

        start_scatter(bi, q)
        return 0

    lax.fori_loop(0, nblocks, body, 0)

    @pl.when(nblocks > 0)
    def _():
        wait_scatter()

    plsc.subcore_barrier()

    @pl.when(cid == 0)
    def _():
        pltpu.sync_copy(acc.at[pl.ds(base, ROWS_PT)], outa.at[pl.ds(base, ROWS_PT)])

    @pl.when(cid == 1)
    def _():
        pltpu.sync_copy(acc.at[pl.ds(base, ROWS_PT)], outb.at[pl.ds(base, ROWS_PT)])


_agg_kernel = functools.partial(
    pl.kernel,
    out_type=(jax.ShapeDtypeStruct((NP, H), jnp.float32),
              jax.ShapeDtypeStruct((NP, H), jnp.float32)),
    mesh=_mesh,
    compiler_params=pltpu.CompilerParams(needs_layout_passes=False),
    scratch_types=[
        pltpu.VMEM((CH2 * BLK2,), jnp.int32),
        pltpu.VMEM((CH2, BLK2), jnp.int32),
        pltpu.VMEM((CH2 * BLK2,), jnp.float32),
        pltpu.VMEM((QD * BLK2, H), jnp.float32),
        pltpu.VMEM_SHARED((NP, H), jnp.float32),
        pltpu.SemaphoreType.DMA,
        pltpu.SemaphoreType.DMA,
    ],
)(_agg_body)


def _sagg_body(src_hbm, dst_hbm, ew_hbm, g2_hbm, out_hbm,
               srcb, dstb, ewb, g2v, msgb, zb, acc):
    cid = lax.axis_index("c")
    sid = lax.axis_index("s")
    wid = sid * NC + cid

    _zero_vec(zb, SL1)
    pltpu.sync_copy(zb, acc.at[pl.ds(sid * SL1, SL1)])
    plsc.subcore_barrier()

    pltpu.sync_copy(src_hbm.at[pl.ds(wid * BPT * BLK, BPT * BLK)], srcb)
    pltpu.sync_copy(dst_hbm.at[pl.ds(wid * BPT, BPT)], dstb)
    pltpu.sync_copy(ew_hbm.at[pl.ds(wid * BPT * BLK, BPT * BLK)], ewb)
    pltpu.sync_copy(g2_hbm, g2v)

    def body(b, _):
        for g in range(8):
            sl = pl.ds(b * BLK + g * LANES, LANES)
            src16 = srcb[sl]
            v16 = plsc.load_gather(g2v, [src16])
            msgb[sl] = ewb[sl] * v16
        pltpu.sync_copy(msgb.at[pl.ds(b * BLK, BLK)], acc.at[dstb.at[b]], add=True)
        return 0
    lax.fori_loop(0, BPT, body, 0)

    plsc.subcore_barrier()
    pltpu.sync_copy(acc.at[pl.ds(sid * SL1, SL1)],
                    out_hbm.at[pl.ds(cid * NP + sid * SL1, SL1)])


_sagg_kernel = functools.partial(
    pl.kernel,
    out_type=jax.ShapeDtypeStruct((NC * NP,), jnp.float32),
    mesh=_mesh,
    compiler_params=pltpu.CompilerParams(needs_layout_passes=False),
    scratch_types=[
        pltpu.VMEM((BPT * BLK,), jnp.int32),
        pltpu.VMEM((BPT, BLK), jnp.int32),
        pltpu.VMEM((BPT * BLK,), jnp.float32),
        pltpu.VMEM((N,), jnp.float32),
        pltpu.VMEM((BPT * BLK,), jnp.float32),
        pltpu.VMEM((SL1,), jnp.float32),
        pltpu.VMEM_SHARED((NP,), jnp.float32),
    ],
)(_sagg_body)


def _dinv_body(degp_ref, dinv_ref):
    deg = degp_ref[0:1, :] + degp_ref[1:2, :] + 1.0
    dinv_ref[...] = jnp.where(deg > 0, lax.rsqrt(deg), 0.0)


def _dinv_kernel(deg_part):
    return pl.pallas_call(
        _dinv_body,
        out_shape=jax.ShapeDtypeStruct((1, NP), jnp.float32),
    )(deg_part)


def _dense1_body(x_ref, w1_ref, dc_ref, g1_ref):
    g1_ref[...] = dc_ref[...] * jnp.dot(x_ref[...], w1_ref[...],
                                        preferred_element_type=jnp.float32)


def _dense1(x, W1, dinv_col):
    return pl.pallas_call(
        _dense1_body,
        out_shape=jax.ShapeDtypeStruct((N, H), jnp.float32),
    )(x, W1, dinv_col)


def _dense2_body(agg_a_ref, agg_b_ref, g1_ref, dc_ref, b1_ref, w2_ref, g2_ref):
    dc = dc_ref[...]
    agg = agg_a_ref[0:N, :] + agg_b_ref[0:N, :]
    h1 = jax.nn.relu(dc * (agg + g1_ref[...]) + b1_ref[...])
    g2_ref[...] = dc * jnp.dot(h1, w2_ref[...],
                               preferred_element_type=jnp.float32)


def _dense2(agg_a, agg_b, g1, dinv_col, b1, W2):
    return pl.pallas_call(
        _dense2_body,
        out_shape=jax.ShapeDtypeStruct((N, 1), jnp.float32),
    )(agg_a, agg_b, g1, dinv_col, b1.reshape(1, H), W2)


def _dense3_body(sp_ref, g2_ref, dinv_ref, b2_ref, out_ref):
    s = sp_ref[0:1, 0:N] + sp_ref[1:2, 0:N]
    out_ref[...] = dinv_ref[0:1, 0:N] * (s + g2_ref[...]) + b2_ref[...]


def _dense3(s_part, g2_row, dinv_row, b2):
    return pl.pallas_call(
        _dense3_body,
        out_shape=jax.ShapeDtypeStruct((1, N), jnp.float32),
    )(s_part, g2_row, dinv_row, b2.reshape(1, 1))


def kernel(x, edge_index, edge_attr, W1, b1, W2, b2):
    src = edge_index[0]
    dst = edge_index[1]
    pad = NBLK_ALLOC * BLK - E
    srcf = jnp.concatenate([src, jnp.zeros((pad,), src.dtype)])
    dstp = jnp.concatenate([dst, jnp.zeros((pad,), dst.dtype)]).reshape(
        NBLK_ALLOC, BLK)
    ewf = jnp.concatenate([edge_attr, jnp.zeros((pad,), edge_attr.dtype)])
    ewp = ewf.reshape(NBLK_ALLOC, BLK)

    dstp64 = dstp.reshape(-1, BLK2)

    deg_part = _deg_kernel(dstp, ewp).reshape(NC, NP)
    dinv_row = _dinv_kernel(deg_part)
    dinv_col = dinv_row[0, :N].reshape(N, 1)
    g1 = _dense1(x, W1, dinv_col)
    agg_a, agg_b = _agg_kernel(g1, srcf, dstp64, ewf)
    g2 = _dense2(agg_a, agg_b, g1, dinv_col, b1, W2)
    s_part = _sagg_kernel(srcf, dstp, ewf, g2.reshape(N)).reshape(NC, NP)
    out_row = _dense3(s_part, g2.reshape(1, N), dinv_row, b2)
    return out_row.reshape(N, 1)


# speedup vs baseline: 1.6561x; 1.6561x over previous
import functools

import jax
import jax.numpy as jnp
from jax import lax
from jax.experimental import pallas as pl
from jax.experimental.pallas import tpu as pltpu
from jax.experimental.pallas import tpu_sc as plsc

N = 10000
E = 320000
D = 128
H = 128

NC = 2
NS = 16
NW = NC * NS
LANES = 16

BLK = 128
BPT = 80
E_PAD = NW * BPT * BLK
NBLK = E_PAD // BLK

CH = 16
NBLK_ALLOC = NBLK + CH

BLK2 = 128
QD = 2
CH2 = 16
NBLK2 = E_PAD // BLK2
BPT2_0 = 144
BPT2_1 = 16

NP = 10240
SL1 = NP // NS
ROWS_PT = NP // NS

_mesh = plsc.VectorSubcoreMesh(core_axis_name="c", subcore_axis_name="s")


def _zero_vec(ref, n):
    def body(i, _):
        ref[pl.ds(i * LANES, LANES)] = jnp.zeros((LANES,), jnp.float32)
        return 0
    lax.fori_loop(0, n // LANES, body, 0, unroll=4)


def _deg_body(dst_hbm, ew_hbm, out_hbm, dstb, ewb, zb, acc):
    cid = lax.axis_index("c")
    sid = lax.axis_index("s")
    wid = sid * NC + cid

    _zero_vec(zb, SL1)
    pltpu.sync_copy(zb, acc.at[pl.ds(sid * SL1, SL1)])
    plsc.subcore_barrier()

    pltpu.sync_copy(dst_hbm.at[pl.ds(wid * BPT, BPT)], dstb)
    pltpu.sync_copy(ew_hbm.at[pl.ds(wid * BPT, BPT)], ewb)

    def body(b, _):
        pltpu.sync_copy(ewb.at[b], acc.at[dstb.at[b]], add=True)
        return 0
    lax.fori_loop(0, BPT, body, 0)

    plsc.subcore_barrier()
    pltpu.sync_copy(acc.at[pl.ds(sid * SL1, SL1)],
                    out_hbm.at[pl.ds(cid * NP + sid * SL1, SL1)])


_deg_kernel = functools.partial(
    pl.kernel,
    out_type=jax.ShapeDtypeStruct((NC * NP,), jnp.float32),
    mesh=_mesh,
    compiler_params=pltpu.CompilerParams(needs_layout_passes=False),
    scratch_types=[
        pltpu.VMEM((BPT, BLK), jnp.int32),
        pltpu.VMEM((BPT, BLK), jnp.float32),
        pltpu.VMEM((SL1,), jnp.float32),
        pltpu.VMEM_SHARED((NP,), jnp.float32),
    ],
)(_deg_body)


def _agg_body(g1a_hbm, src_hbm, dst_hbm, ew_hbm,
              outa, outb, srcb, dstb, ewb, rows, acc, gsem, ssem):
    cid = lax.axis_index("c")
    sid = lax.axis_index("s")
    nblocks = lax.select(cid == 0, BPT2_0, BPT2_1)
    blk_base = lax.select(cid == 0, sid * BPT2_0, NS * BPT2_0 + sid * BPT2_1)

    def zrow(r, _):
        for c in range(8):
            rows[r, pl.ds(c * LANES, LANES)] = jnp.zeros((LANES,), jnp.float32)
        return 0
    lax.fori_loop(0, 128, zrow, 0)
    base = sid * ROWS_PT
    for off in (0, 128, 256, 384, 512):
        pltpu.sync_copy(rows.at[pl.ds(0, 128)], acc.at[pl.ds(base + off, 128)])
    plsc.subcore_barrier()

    def start_gather(bi, q):
        pltpu.async_copy(g1a_hbm.at[srcb.at[pl.ds(bi * BLK2, BLK2)]],
                         rows.at[pl.ds(q * BLK2, BLK2)], gsem)

    def wait_gather():
        pltpu.make_async_copy(g1a_hbm.at[srcb.at[pl.ds(0, BLK2)]],
                              rows.at[pl.ds(0, BLK2)], gsem).wait()

    def start_scatter(bi, q):
        pltpu.async_copy(rows.at[pl.ds(q * BLK2, BLK2)],
                         acc.at[dstb.at[bi]], ssem, add=True)

    def wait_scatter():
        pltpu.make_async_copy(rows.at[pl.ds(0, BLK2)],
                              acc.at[dstb.at[0]], ssem).wait()

    def refill(ci):
        off = blk_base + ci * CH2
        pltpu.sync_copy(src_hbm.at[pl.ds(off * BLK2, CH2 * BLK2)], srcb)
        pltpu.sync_copy(dst_hbm.at[pl.ds(off, CH2)], dstb)
        pltpu.sync_copy(ew_hbm.at[pl.ds(off * BLK2, CH2 * BLK2)], ewb)

    def body(b, _):
        bi = lax.rem(b, CH2)
        q = lax.rem(b, QD)

        @pl.when(bi == 0)
        def _():
            @pl.when(b > 0)
            def _():
                wait_scatter()
            refill(lax.div(b, CH2))
            for p in range(QD - 1):
                @pl.when(b + p < nblocks)
                def _():
                    start_gather(p, lax.rem(b + p, QD))

        wait_gather()

        @pl.when((bi != 0) & (b > 0))
        def _():
            wait_scatter()

        @pl.when((bi + QD - 1 < CH2) & (b + QD - 1 < nblocks))
        def _():
            start_gather(bi + QD - 1, lax.rem(b + QD - 1, QD))

        def scale(rr, _):
            bc = plsc.load_gather(
                ewb, [jnp.full((LANES,), bi * BLK2 + rr, jnp.int32)])
            row = q * BLK2 + rr
            for c in range(8):
                sl = pl.ds(c * LANES, LANES)
                rows[row, sl] = rows[row, sl] * bc
            return 0
        lax.fori_loop(0, BLK2, scale, 0, unroll=4)

        start_scatter(bi, q)
        return 0

    lax.fori_loop(0, nblocks, body, 0)

    @pl.when(nblocks > 0)
    def _():
        wait_scatter()

    plsc.subcore_barrier()

    @pl.when(cid == 0)
    def _():
        pltpu.sync_copy(acc.at[pl.ds(base, ROWS_PT)], outa.at[pl.ds(base, ROWS_PT)])

    @pl.when(cid == 1)
    def _():
        pltpu.sync_copy(acc.at[pl.ds(base, ROWS_PT)], outb.at[pl.ds(base, ROWS_PT)])


_agg_kernel = functools.partial(
    pl.kernel,
    out_type=(jax.ShapeDtypeStruct((NP, H), jnp.float32),
              jax.ShapeDtypeStruct((NP, H), jnp.float32)),
    mesh=_mesh,
    compiler_params=pltpu.CompilerParams(needs_layout_passes=False),
    scratch_types=[
        pltpu.VMEM((CH2 * BLK2,), jnp.int32),
        pltpu.VMEM((CH2, BLK2), jnp.int32),
        pltpu.VMEM((CH2 * BLK2,), jnp.float32),
        pltpu.VMEM((QD * BLK2, H), jnp.float32),
        pltpu.VMEM_SHARED((NP, H), jnp.float32),
        pltpu.SemaphoreType.DMA,
        pltpu.SemaphoreType.DMA,
    ],
)(_agg_body)


def _sagg_body(src_hbm, dst_hbm, ew_hbm, g2_hbm, out_hbm,
               srcb, dstb, ewb, g2v, msgb, zb, acc):
    cid = lax.axis_index("c")
    sid = lax.axis_index("s")
    wid = sid * NC + cid

    _zero_vec(zb, SL1)
    pltpu.sync_copy(zb, acc.at[pl.ds(sid * SL1, SL1)])
    plsc.subcore_barrier()

    pltpu.sync_copy(src_hbm.at[pl.ds(wid * BPT * BLK, BPT * BLK)], srcb)
    pltpu.sync_copy(dst_hbm.at[pl.ds(wid * BPT, BPT)], dstb)
    pltpu.sync_copy(ew_hbm.at[pl.ds(wid * BPT * BLK, BPT * BLK)], ewb)
    pltpu.sync_copy(g2_hbm, g2v)

    def body(b, _):
        for g in range(8):
            sl = pl.ds(b * BLK + g * LANES, LANES)
            src16 = srcb[sl]
            v16 = plsc.load_gather(g2v, [src16])
            msgb[sl] = ewb[sl] * v16
        pltpu.sync_copy(msgb.at[pl.ds(b * BLK, BLK)], acc.at[dstb.at[b]], add=True)
        return 0
    lax.fori_loop(0, BPT, body, 0)

    plsc.subcore_barrier()
    pltpu.sync_copy(acc.at[pl.ds(sid * SL1, SL1)],
                    out_hbm.at[pl.ds(cid * NP + sid * SL1, SL1)])


_sagg_kernel = functools.partial(
    pl.kernel,
    out_type=jax.ShapeDtypeStruct((NC * NP,), jnp.float32),
    mesh=_mesh,
    compiler_params=pltpu.CompilerParams(needs_layout_passes=False),
    scratch_types=[
        pltpu.VMEM((BPT * BLK,), jnp.int32),
        pltpu.VMEM((BPT, BLK), jnp.int32),
        pltpu.VMEM((BPT * BLK,), jnp.float32),
        pltpu.VMEM((N,), jnp.float32),
        pltpu.VMEM((BPT * BLK,), jnp.float32),
        pltpu.VMEM((SL1,), jnp.float32),
        pltpu.VMEM_SHARED((NP,), jnp.float32),
    ],
)(_sagg_body)


def _dinv_body(degp_ref, dinv_ref):
    deg = degp_ref[0:1, :] + degp_ref[1:2, :] + 1.0
    dinv_ref[...] = jnp.where(deg > 0, lax.rsqrt(deg), 0.0)


def _dinv_kernel(deg_part):
    return pl.pallas_call(
        _dinv_body,
        out_shape=jax.ShapeDtypeStruct((1, NP), jnp.float32),
    )(deg_part)


def _dense1_body(x_ref, w1_ref, dc_ref, g1_ref):
    g1_ref[...] = dc_ref[...] * jnp.dot(x_ref[...], w1_ref[...],
                                        preferred_element_type=jnp.float32)


def _dense1(x, W1, dinv_col):
    return pl.pallas_call(
        _dense1_body,
        out_shape=jax.ShapeDtypeStruct((N, H), jnp.float32),
    )(x, W1, dinv_col)


def _dense2_body(agg_a_ref, agg_b_ref, g1_ref, dc_ref, b1_ref, w2_ref, g2_ref):
    dc = dc_ref[...]
    agg = agg_a_ref[0:N, :] + agg_b_ref[0:N, :]
    h1 = jax.nn.relu(dc * (agg + g1_ref[...]) + b1_ref[...])
    g2_ref[...] = dc * jnp.dot(h1, w2_ref[...],
                               preferred_element_type=jnp.float32)


def _dense2(agg_a, agg_b, g1, dinv_col, b1, W2):
    return pl.pallas_call(
        _dense2_body,
        out_shape=jax.ShapeDtypeStruct((N, 1), jnp.float32),
    )(agg_a, agg_b, g1, dinv_col, b1.reshape(1, H), W2)


def _dense3_body(sp_ref, g2_ref, dinv_ref, b2_ref, out_ref):
    s = sp_ref[0:1, 0:N] + sp_ref[1:2, 0:N]
    out_ref[...] = dinv_ref[0:1, 0:N] * (s + g2_ref[...]) + b2_ref[...]


def _dense3(s_part, g2_row, dinv_row, b2):
    return pl.pallas_call(
        _dense3_body,
        out_shape=jax.ShapeDtypeStruct((1, N), jnp.float32),
    )(s_part, g2_row, dinv_row, b2.reshape(1, 1))


def kernel(x, edge_index, edge_attr, W1, b1, W2, b2):
    src = edge_index[0]
    dst = edge_index[1]
    pad = NBLK_ALLOC * BLK - E
    srcf = jnp.concatenate([src, jnp.zeros((pad,), src.dtype)])
    dstp = jnp.concatenate([dst, jnp.zeros((pad,), dst.dtype)]).reshape(
        NBLK_ALLOC, BLK)
    ewf = jnp.concatenate([edge_attr, jnp.zeros((pad,), edge_attr.dtype)])
    ewp = ewf.reshape(NBLK_ALLOC, BLK)

    dstp64 = dstp.reshape(-1, BLK2)

    deg_part = _deg_kernel(dstp, ewp).reshape(NC, NP)
    dinv_row = _dinv_kernel(deg_part)
    dinv_col = dinv_row[0, :N].reshape(N, 1)
    g1 = _dense1(x, W1, dinv_col)
    agg_a, agg_b = _agg_kernel(g1, srcf, dstp64, ewf)
    g2 = _dense2(agg_a, agg_b, g1, dinv_col, b1, W2)
    s_part = _sagg_kernel(srcf, dstp, ewf, g2.reshape(N)).reshape(NC, NP)
    out_row = _dense3(s_part, g2.reshape(1, N), dinv_row, b2)
    return out_row.reshape(N, 1)
